# Initial kernel scaffold; baseline (speedup 1.0000x reference)
import functools

import jax
import jax.numpy as jnp
from jax import lax
from jax.experimental import pallas as pl
from jax.experimental.pallas import tpu as pltpu
from jax.experimental.pallas import tpu_sc as plsc

_N_NODES = 10000
_N_EDGES = 160000
_EDGE_DIM = 16
_HI_B = 41
_FEAT = 32
_EHID = 32
_DP = 48
_KN = _EHID * _FEAT
_NC = 2
_NS = 16
_NW = _NC * _NS
_EPT = _N_EDGES // _NW
_GC = 1000
_NCH = _EPT // _GC
_RPT = _N_NODES // _NS

_BE = 2000
_BN = 2000


def _gather_body(x1_hbm, src_hbm, out_hbm, idx_v, rows_v, sem):
    cid = lax.axis_index("c")
    sid = lax.axis_index("s")
    wid = sid * _NC + cid
    for j in range(_NCH):
        base = wid * _EPT + j * _GC
        pltpu.sync_copy(src_hbm.at[pl.ds(base, _GC)], idx_v)
        pltpu.async_copy(x1_hbm.at[idx_v], rows_v, sem).wait()
        pltpu.sync_copy(rows_v, out_hbm.at[pl.ds(base, _GC)])


def _sc_gather(x1p, src):
    mesh = plsc.VectorSubcoreMesh(core_axis_name="c", subcore_axis_name="s")
    run = pl.kernel(
        _gather_body,
        out_type=jax.ShapeDtypeStruct((_N_EDGES, _DP), jnp.float32),
        mesh=mesh,
        scratch_types=[
            pltpu.VMEM((_GC,), jnp.int32),
            pltpu.VMEM((_GC, _DP), jnp.float32),
            pltpu.SemaphoreType.DMA,
        ],
    )
    return run(x1p, src)


def _scatter_body(msg_hbm, dst_hbm, zero_hbm, out_hbm, idx_v, rows_v, table_sh):
    cid = lax.axis_index("c")
    sid = lax.axis_index("s")
    wid = sid * _NC + cid
    pltpu.sync_copy(zero_hbm.at[pl.ds(sid * _RPT, _RPT)],
                    table_sh.at[pl.ds(sid * _RPT, _RPT)])
    plsc.subcore_barrier()
    for j in range(_NCH):
        base = wid * _EPT + j * _GC
        pltpu.sync_copy(dst_hbm.at[pl.ds(base, _GC)], idx_v)
        pltpu.sync_copy(msg_hbm.at[pl.ds(base, _GC)], rows_v)
        pltpu.sync_copy(rows_v, table_sh.at[idx_v], add=True)
    plsc.subcore_barrier()
    pltpu.sync_copy(table_sh.at[pl.ds(sid * _RPT, _RPT)],
                    out_hbm.at[cid, pl.ds(sid * _RPT, _RPT)])


def _sc_scatter(msg, dst, zeros):
    mesh = plsc.VectorSubcoreMesh(core_axis_name="c", subcore_axis_name="s")
    run = pl.kernel(
        _scatter_body,
        out_type=jax.ShapeDtypeStruct((_NC, _N_NODES, _DP), jnp.float32),
        mesh=mesh,
        scratch_types=[
            pltpu.VMEM((_GC,), jnp.int32),
            pltpu.VMEM((_GC, _DP), jnp.float32),
            pltpu.VMEM_SHARED((_N_NODES, _DP), jnp.float32),
        ],
    )
    return run(msg, dst, zeros)


def _edge_body(ea_ref, xs_ref, w1_ref, b1_ref, w2_ref, out_ref):
    h = jnp.maximum(
        jnp.dot(ea_ref[...], w1_ref[...], preferred_element_type=jnp.float32)
        + b1_ref[...], 0.0)
    G = jnp.dot(xs_ref[...], w2_ref[...], preferred_element_type=jnp.float32)
    acc = G[:, _KN:_KN + _FEAT]
    for k in range(_EHID):
        acc = acc + h[:, k:k + 1] * G[:, k * _FEAT:(k + 1) * _FEAT]
    out_ref[:, :_FEAT] = acc
    ones_col = (lax.broadcasted_iota(jnp.int32, (_BE, _DP - _FEAT), 1) == 0
                ).astype(jnp.float32)
    out_ref[:, _FEAT:] = ones_col


def _tc_edges(ea, xs, w1, b1, w2aug):
    grid = _N_EDGES // _BE
    return pl.pallas_call(
        _edge_body,
        grid=(grid,),
        in_specs=[
            pl.BlockSpec((_BE, _EDGE_DIM), lambda i: (i, 0)),
            pl.BlockSpec((_BE, _DP), lambda i: (i, 0)),
            pl.BlockSpec((_EDGE_DIM, _EHID), lambda i: (0, 0)),
            pl.BlockSpec((1, _EHID), lambda i: (0, 0)),
            pl.BlockSpec((_DP, _KN + _FEAT), lambda i: (0, 0)),
        ],
        out_specs=pl.BlockSpec((_BE, _DP), lambda i: (i, 0)),
        out_shape=jax.ShapeDtypeStruct((_N_EDGES, _DP), jnp.float32),
    )(ea, xs, w1, b1, w2aug)


def _node_body(x1_ref, p0_ref, p1_ref, rw_ref, b_ref, out_ref):
    s = p0_ref[...] + p1_ref[...]
    agg = s[:, :_FEAT] / jnp.maximum(s[:, _FEAT:_FEAT + 1], 1.0)
    out = (jnp.dot(x1_ref[...], rw_ref[...], preferred_element_type=jnp.float32)
           + agg + b_ref[...])
    out_ref[...] = jnp.maximum(out, 0.0)


def _tc_nodes(x1p, p0, p1, rootp, bias2):
    grid = _N_NODES // _BN
    return pl.pallas_call(
        _node_body,
        grid=(grid,),
        in_specs=[
            pl.BlockSpec((_BN, _DP), lambda i: (i, 0)),
            pl.BlockSpec((_BN, _DP), lambda i: (i, 0)),
            pl.BlockSpec((_BN, _DP), lambda i: (i, 0)),
            pl.BlockSpec((_DP, _FEAT), lambda i: (0, 0)),
            pl.BlockSpec((1, _FEAT), lambda i: (0, 0)),
        ],
        out_specs=pl.BlockSpec((_BN, _FEAT), lambda i: (i, 0)),
        out_shape=jax.ShapeDtypeStruct((_N_NODES, _FEAT), jnp.float32),
    )(x1p, p0, p1, rootp, bias2)


@jax.jit
def kernel(x, edge_index, edge_attr, lin1_w, lin1_b, lin2_w, lin2_b,
           root_w, bias):
    x1p = jnp.pad(x[:, :_HI_B], ((0, 0), (0, _DP - _HI_B)))
    src = edge_index[0]
    dst = edge_index[1]
    w2r = jnp.pad(
        lin2_w.reshape(_EHID, _HI_B, _FEAT).transpose(1, 0, 2)
        .reshape(_HI_B, _KN), ((0, _DP - _HI_B), (0, 0)))
    b2p = jnp.pad(lin2_b.reshape(_HI_B, _FEAT), ((0, _DP - _HI_B), (0, 0)))
    w2aug = jnp.concatenate([w2r, b2p], axis=1)
    rootp = jnp.pad(root_w, ((0, _DP - _HI_B), (0, 0)))

    x_src = _sc_gather(x1p, src)
    msg = _tc_edges(edge_attr, x_src, lin1_w, lin1_b.reshape(1, -1), w2aug)
    parts = _sc_scatter(msg, dst, jnp.zeros((_N_NODES, _DP), jnp.float32))
    return _tc_nodes(x1p, parts[0], parts[1], rootp, bias.reshape(1, -1))


# speedup vs baseline: 1.5720x; 1.5720x over previous
import functools

import jax
import jax.numpy as jnp
from jax import lax
from jax.experimental import pallas as pl
from jax.experimental.pallas import tpu as pltpu
from jax.experimental.pallas import tpu_sc as plsc

_N_NODES = 10000
_N_EDGES = 160000
_EDGE_DIM = 16
_HI_B = 41
_FEAT = 32
_EHID = 32
_DP = 48
_KN = _EHID * _FEAT
_NC = 2
_NS = 16
_NW = _NC * _NS
_EPT = _N_EDGES // _NW
_GC = 1000
_NCH = _EPT // _GC
_RPT = _N_NODES // _NS

_BE = 2000
_BN = 2000


def _gather_body(x1_hbm, src_hbm, out_hbm, idx_v, rows_v, sem):
    cid = lax.axis_index("c")
    sid = lax.axis_index("s")
    wid = sid * _NC + cid
    for j in range(_NCH):
        base = wid * _EPT + j * _GC
        pltpu.sync_copy(src_hbm.at[pl.ds(base, _GC)], idx_v)
        pltpu.async_copy(x1_hbm.at[idx_v], rows_v, sem).wait()
        pltpu.sync_copy(rows_v, out_hbm.at[pl.ds(base, _GC)])


def _sc_gather(x1p, src):
    mesh = plsc.VectorSubcoreMesh(core_axis_name="c", subcore_axis_name="s")
    run = pl.kernel(
        _gather_body,
        out_type=jax.ShapeDtypeStruct((_N_EDGES, _DP), jnp.float32),
        mesh=mesh,
        scratch_types=[
            pltpu.VMEM((_GC,), jnp.int32),
            pltpu.VMEM((_GC, _DP), jnp.float32),
            pltpu.SemaphoreType.DMA,
        ],
        compiler_params=pltpu.CompilerParams(use_tc_tiling_on_sc=False),
    )
    return run(x1p, src)


def _scatter_body(msg_hbm, dst_hbm, zero_hbm, out_hbm, idx_v, rows_v, table_sh):
    cid = lax.axis_index("c")
    sid = lax.axis_index("s")
    wid = sid * _NC + cid
    pltpu.sync_copy(zero_hbm.at[pl.ds(sid * _RPT, _RPT)],
                    table_sh.at[pl.ds(sid * _RPT, _RPT)])
    plsc.subcore_barrier()
    for j in range(_NCH):
        base = wid * _EPT + j * _GC
        pltpu.sync_copy(dst_hbm.at[pl.ds(base, _GC)], idx_v)
        pltpu.sync_copy(msg_hbm.at[pl.ds(base, _GC)], rows_v)
        pltpu.sync_copy(rows_v, table_sh.at[idx_v], add=True)
    plsc.subcore_barrier()
    pltpu.sync_copy(table_sh.at[pl.ds(sid * _RPT, _RPT)],
                    out_hbm.at[cid, pl.ds(sid * _RPT, _RPT)])


def _sc_scatter(msg, dst, zeros):
    mesh = plsc.VectorSubcoreMesh(core_axis_name="c", subcore_axis_name="s")
    run = pl.kernel(
        _scatter_body,
        out_type=jax.ShapeDtypeStruct((_NC, _N_NODES, _DP), jnp.float32),
        mesh=mesh,
        scratch_types=[
            pltpu.VMEM((_GC,), jnp.int32),
            pltpu.VMEM((_GC, _DP), jnp.float32),
            pltpu.VMEM_SHARED((_N_NODES, _DP), jnp.float32),
        ],
        compiler_params=pltpu.CompilerParams(use_tc_tiling_on_sc=False),
    )
    return run(msg, dst, zeros)


def _edge_body(ea_ref, xs_ref, w1_ref, b1_ref, w2_ref, out_ref):
    h = jnp.maximum(
        jnp.dot(ea_ref[...], w1_ref[...], preferred_element_type=jnp.float32)
        + b1_ref[...], 0.0)
    G = jnp.dot(xs_ref[...], w2_ref[...], preferred_element_type=jnp.float32)
    acc = G[:, _KN:_KN + _FEAT]
    for k in range(_EHID):
        acc = acc + h[:, k:k + 1] * G[:, k * _FEAT:(k + 1) * _FEAT]
    out_ref[:, :_FEAT] = acc
    ones_col = (lax.broadcasted_iota(jnp.int32, (_BE, _DP - _FEAT), 1) == 0
                ).astype(jnp.float32)
    out_ref[:, _FEAT:] = ones_col


def _tc_edges(ea, xs, w1, b1, w2aug):
    grid = _N_EDGES // _BE
    return pl.pallas_call(
        _edge_body,
        grid=(grid,),
        in_specs=[
            pl.BlockSpec((_BE, _EDGE_DIM), lambda i: (i, 0)),
            pl.BlockSpec((_BE, _DP), lambda i: (i, 0)),
            pl.BlockSpec((_EDGE_DIM, _EHID), lambda i: (0, 0)),
            pl.BlockSpec((1, _EHID), lambda i: (0, 0)),
            pl.BlockSpec((_DP, _KN + _FEAT), lambda i: (0, 0)),
        ],
        out_specs=pl.BlockSpec((_BE, _DP), lambda i: (i, 0)),
        out_shape=jax.ShapeDtypeStruct((_N_EDGES, _DP), jnp.float32),
    )(ea, xs, w1, b1, w2aug)


def _node_body(x1_ref, p0_ref, p1_ref, rw_ref, b_ref, out_ref):
    s = p0_ref[...] + p1_ref[...]
    agg = s[:, :_FEAT] / jnp.maximum(s[:, _FEAT:_FEAT + 1], 1.0)
    out = (jnp.dot(x1_ref[...], rw_ref[...], preferred_element_type=jnp.float32)
           + agg + b_ref[...])
    out_ref[...] = jnp.maximum(out, 0.0)


def _tc_nodes(x1p, p0, p1, rootp, bias2):
    grid = _N_NODES // _BN
    return pl.pallas_call(
        _node_body,
        grid=(grid,),
        in_specs=[
            pl.BlockSpec((_BN, _DP), lambda i: (i, 0)),
            pl.BlockSpec((_BN, _DP), lambda i: (i, 0)),
            pl.BlockSpec((_BN, _DP), lambda i: (i, 0)),
            pl.BlockSpec((_DP, _FEAT), lambda i: (0, 0)),
            pl.BlockSpec((1, _FEAT), lambda i: (0, 0)),
        ],
        out_specs=pl.BlockSpec((_BN, _FEAT), lambda i: (i, 0)),
        out_shape=jax.ShapeDtypeStruct((_N_NODES, _FEAT), jnp.float32),
    )(x1p, p0, p1, rootp, bias2)


@jax.jit
def kernel(x, edge_index, edge_attr, lin1_w, lin1_b, lin2_w, lin2_b,
           root_w, bias):
    x1p = jnp.pad(x[:, :_HI_B], ((0, 0), (0, _DP - _HI_B)))
    src = edge_index[0]
    dst = edge_index[1]
    w2r = jnp.pad(
        lin2_w.reshape(_EHID, _HI_B, _FEAT).transpose(1, 0, 2)
        .reshape(_HI_B, _KN), ((0, _DP - _HI_B), (0, 0)))
    b2p = jnp.pad(lin2_b.reshape(_HI_B, _FEAT), ((0, _DP - _HI_B), (0, 0)))
    w2aug = jnp.concatenate([w2r, b2p], axis=1)
    rootp = jnp.pad(root_w, ((0, _DP - _HI_B), (0, 0)))

    x_src = _sc_gather(x1p, src)
    msg = _tc_edges(edge_attr, x_src, lin1_w, lin1_b.reshape(1, -1), w2aug)
    parts = _sc_scatter(msg, dst, jnp.zeros((_N_NODES, _DP), jnp.float32))
    return _tc_nodes(x1p, parts[0], parts[1], rootp, bias.reshape(1, -1))


# speedup vs baseline: 5.3940x; 3.4313x over previous
import functools

import jax
import jax.numpy as jnp
from jax import lax
from jax.experimental import pallas as pl
from jax.experimental.pallas import tpu as pltpu
from jax.experimental.pallas import tpu_sc as plsc

_N_NODES = 10000
_N_EDGES = 160000
_EDGE_DIM = 16
_HI_B = 41
_FEAT = 32
_EHID = 32
_DP = 48
_KN = _EHID * _FEAT
_NC = 2
_NS = 16
_NW = _NC * _NS
_EPT = _N_EDGES // _NW
_GC = 1000
_NCH = _EPT // _GC
_RPT = _N_NODES // _NS

_BE = 1280
_BN = 2000


def _gather_body(x1_hbm, src_hbm, out_hbm, idx_v, rows_v, sem):
    cid = lax.axis_index("c")
    sid = lax.axis_index("s")
    wid = sid * _NC + cid
    for j in range(_NCH):
        base = wid * _EPT + j * _GC
        pltpu.sync_copy(src_hbm.at[pl.ds(base, _GC)], idx_v)
        pltpu.async_copy(x1_hbm.at[idx_v], rows_v, sem).wait()
        pltpu.sync_copy(rows_v, out_hbm.at[pl.ds(base, _GC)])


def _sc_gather(x1p, src):
    mesh = plsc.VectorSubcoreMesh(core_axis_name="c", subcore_axis_name="s")
    run = pl.kernel(
        _gather_body,
        out_type=jax.ShapeDtypeStruct((_N_EDGES, _DP), jnp.float32),
        mesh=mesh,
        scratch_types=[
            pltpu.VMEM((_GC,), jnp.int32),
            pltpu.VMEM((_GC, _DP), jnp.float32),
            pltpu.SemaphoreType.DMA,
        ],
        compiler_params=pltpu.CompilerParams(use_tc_tiling_on_sc=False),
    )
    return run(x1p, src)


def _scatter_body(msg_hbm, dst_hbm, zero_hbm, out_hbm, idx_v, rows_v, table_sh):
    cid = lax.axis_index("c")
    sid = lax.axis_index("s")
    wid = sid * _NC + cid
    pltpu.sync_copy(zero_hbm.at[pl.ds(sid * _RPT, _RPT)],
                    table_sh.at[pl.ds(sid * _RPT, _RPT)])
    plsc.subcore_barrier()
    for j in range(_NCH):
        base = wid * _EPT + j * _GC
        pltpu.sync_copy(dst_hbm.at[pl.ds(base, _GC)], idx_v)
        pltpu.sync_copy(msg_hbm.at[pl.ds(base, _GC)], rows_v)
        pltpu.sync_copy(rows_v, table_sh.at[idx_v], add=True)
    plsc.subcore_barrier()
    pltpu.sync_copy(table_sh.at[pl.ds(sid * _RPT, _RPT)],
                    out_hbm.at[cid, pl.ds(sid * _RPT, _RPT)])


def _sc_scatter(msg, dst, zeros):
    mesh = plsc.VectorSubcoreMesh(core_axis_name="c", subcore_axis_name="s")
    run = pl.kernel(
        _scatter_body,
        out_type=jax.ShapeDtypeStruct((_NC, _N_NODES, _DP), jnp.float32),
        mesh=mesh,
        scratch_types=[
            pltpu.VMEM((_GC,), jnp.int32),
            pltpu.VMEM((_GC, _DP), jnp.float32),
            pltpu.VMEM_SHARED((_N_NODES, _DP), jnp.float32),
        ],
        compiler_params=pltpu.CompilerParams(use_tc_tiling_on_sc=False),
    )
    return run(msg, dst, zeros)


def _edge_body(eat_ref, xs_ref, w1_ref, b1t_ref, w2_ref, out_ref):
    h_t = jnp.maximum(
        lax.dot_general(w1_ref[...], eat_ref[...], (((0,), (0,)), ((), ())),
                        preferred_element_type=jnp.float32)
        + b1t_ref[...], 0.0)
    G_t = lax.dot_general(w2_ref[...], xs_ref[...], (((0,), (1,)), ((), ())),
                          preferred_element_type=jnp.float32)
    acc = G_t[_KN:_KN + _FEAT, :]
    for k in range(_EHID):
        acc = acc + h_t[k:k + 1, :] * G_t[k * _FEAT:(k + 1) * _FEAT, :]
    out_ref[:, :_FEAT] = acc.T
    ones_col = (lax.broadcasted_iota(jnp.int32, (_BE, _DP - _FEAT), 1) == 0
                ).astype(jnp.float32)
    out_ref[:, _FEAT:] = ones_col


def _tc_edges(eat, xs, w1, b1t, w2aug):
    grid = _N_EDGES // _BE
    return pl.pallas_call(
        _edge_body,
        grid=(grid,),
        in_specs=[
            pl.BlockSpec((_EDGE_DIM, _BE), lambda i: (0, i)),
            pl.BlockSpec((_BE, _DP), lambda i: (i, 0)),
            pl.BlockSpec((_EDGE_DIM, _EHID), lambda i: (0, 0)),
            pl.BlockSpec((_EHID, 1), lambda i: (0, 0)),
            pl.BlockSpec((_DP, _KN + _FEAT), lambda i: (0, 0)),
        ],
        out_specs=pl.BlockSpec((_BE, _DP), lambda i: (i, 0)),
        out_shape=jax.ShapeDtypeStruct((_N_EDGES, _DP), jnp.float32),
    )(eat, xs, w1, b1t, w2aug)


def _node_body(x1_ref, p0_ref, p1_ref, rw_ref, b_ref, out_ref):
    s = p0_ref[...] + p1_ref[...]
    agg = s[:, :_FEAT] / jnp.maximum(s[:, _FEAT:_FEAT + 1], 1.0)
    out = (jnp.dot(x1_ref[...], rw_ref[...], preferred_element_type=jnp.float32)
           + agg + b_ref[...])
    out_ref[...] = jnp.maximum(out, 0.0)


def _tc_nodes(x1p, p0, p1, rootp, bias2):
    grid = _N_NODES // _BN
    return pl.pallas_call(
        _node_body,
        grid=(grid,),
        in_specs=[
            pl.BlockSpec((_BN, _DP), lambda i: (i, 0)),
            pl.BlockSpec((_BN, _DP), lambda i: (i, 0)),
            pl.BlockSpec((_BN, _DP), lambda i: (i, 0)),
            pl.BlockSpec((_DP, _FEAT), lambda i: (0, 0)),
            pl.BlockSpec((1, _FEAT), lambda i: (0, 0)),
        ],
        out_specs=pl.BlockSpec((_BN, _FEAT), lambda i: (i, 0)),
        out_shape=jax.ShapeDtypeStruct((_N_NODES, _FEAT), jnp.float32),
    )(x1p, p0, p1, rootp, bias2)


@jax.jit
def kernel(x, edge_index, edge_attr, lin1_w, lin1_b, lin2_w, lin2_b,
           root_w, bias):
    x1p = jnp.pad(x[:, :_HI_B], ((0, 0), (0, _DP - _HI_B)))
    src = edge_index[0]
    dst = edge_index[1]
    w2r = jnp.pad(
        lin2_w.reshape(_EHID, _HI_B, _FEAT).transpose(1, 0, 2)
        .reshape(_HI_B, _KN), ((0, _DP - _HI_B), (0, 0)))
    b2p = jnp.pad(lin2_b.reshape(_HI_B, _FEAT), ((0, _DP - _HI_B), (0, 0)))
    w2aug = jnp.concatenate([w2r, b2p], axis=1)
    rootp = jnp.pad(root_w, ((0, _DP - _HI_B), (0, 0)))

    x_src = _sc_gather(x1p, src)
    msg = _tc_edges(edge_attr.T, x_src, lin1_w, lin1_b.reshape(-1, 1), w2aug)
    parts = _sc_scatter(msg, dst, jnp.zeros((_N_NODES, _DP), jnp.float32))
    return _tc_nodes(x1p, parts[0], parts[1], rootp, bias.reshape(1, -1))


# speedup vs baseline: 5.5567x; 1.0301x over previous
import functools

import jax
import jax.numpy as jnp
from jax import lax
from jax.experimental import pallas as pl
from jax.experimental.pallas import tpu as pltpu
from jax.experimental.pallas import tpu_sc as plsc

_N_NODES = 10000
_N_EDGES = 160000
_EDGE_DIM = 16
_HI_B = 41
_FEAT = 32
_EHID = 32
_DP = 48
_KN = _EHID * _FEAT
_NC = 2
_NS = 16
_NW = _NC * _NS
_EPT = _N_EDGES // _NW
_GC = 1000
_NCH = _EPT // _GC
_RPT = _N_NODES // _NS

_BE = 3200
_BN = 2000


def _gather_body(x1_hbm, src_hbm, out_hbm, idx_v, rows_v, sem):
    cid = lax.axis_index("c")
    sid = lax.axis_index("s")
    wid = sid * _NC + cid
    for j in range(_NCH):
        base = wid * _EPT + j * _GC
        pltpu.sync_copy(src_hbm.at[pl.ds(base, _GC)], idx_v)
        pltpu.async_copy(x1_hbm.at[idx_v], rows_v, sem).wait()
        pltpu.sync_copy(rows_v, out_hbm.at[pl.ds(base, _GC)])


def _sc_gather(x1p, src):
    mesh = plsc.VectorSubcoreMesh(core_axis_name="c", subcore_axis_name="s")
    run = pl.kernel(
        _gather_body,
        out_type=jax.ShapeDtypeStruct((_N_EDGES, _DP), jnp.float32),
        mesh=mesh,
        scratch_types=[
            pltpu.VMEM((_GC,), jnp.int32),
            pltpu.VMEM((_GC, _DP), jnp.float32),
            pltpu.SemaphoreType.DMA,
        ],
        compiler_params=pltpu.CompilerParams(use_tc_tiling_on_sc=False),
    )
    return run(x1p, src)


def _scatter_body(msg_hbm, dst_hbm, zero_hbm, out_hbm, idx_v, rows_v, table_sh):
    cid = lax.axis_index("c")
    sid = lax.axis_index("s")
    wid = sid * _NC + cid
    pltpu.sync_copy(zero_hbm.at[pl.ds(sid * _RPT, _RPT)],
                    table_sh.at[pl.ds(sid * _RPT, _RPT)])
    plsc.subcore_barrier()
    for j in range(_NCH):
        base = wid * _EPT + j * _GC
        pltpu.sync_copy(dst_hbm.at[pl.ds(base, _GC)], idx_v)
        pltpu.sync_copy(msg_hbm.at[pl.ds(base, _GC)], rows_v)
        pltpu.sync_copy(rows_v, table_sh.at[idx_v], add=True)
    plsc.subcore_barrier()
    pltpu.sync_copy(table_sh.at[pl.ds(sid * _RPT, _RPT)],
                    out_hbm.at[cid, pl.ds(sid * _RPT, _RPT)])


def _sc_scatter(msg, dst, zeros):
    mesh = plsc.VectorSubcoreMesh(core_axis_name="c", subcore_axis_name="s")
    run = pl.kernel(
        _scatter_body,
        out_type=jax.ShapeDtypeStruct((_NC, _N_NODES, _DP), jnp.float32),
        mesh=mesh,
        scratch_types=[
            pltpu.VMEM((_GC,), jnp.int32),
            pltpu.VMEM((_GC, _DP), jnp.float32),
            pltpu.VMEM_SHARED((_N_NODES, _DP), jnp.float32),
        ],
        compiler_params=pltpu.CompilerParams(use_tc_tiling_on_sc=False),
    )
    return run(msg, dst, zeros)


def _edge_body(ea_ref, xs_ref, w1_ref, b1t_ref, w2_ref, out_ref):
    h_t = jnp.maximum(
        lax.dot_general(w1_ref[...], ea_ref[...], (((0,), (1,)), ((), ())),
                        preferred_element_type=jnp.float32)
        + b1t_ref[...], 0.0)
    G_t = lax.dot_general(w2_ref[...], xs_ref[...], (((0,), (1,)), ((), ())),
                          preferred_element_type=jnp.float32)
    acc = G_t[_KN:_KN + _FEAT, :]
    for k in range(_EHID):
        acc = acc + h_t[k:k + 1, :] * G_t[k * _FEAT:(k + 1) * _FEAT, :]
    out_ref[:, :_FEAT] = acc.T
    ones_col = (lax.broadcasted_iota(jnp.int32, (_BE, _DP - _FEAT), 1) == 0
                ).astype(jnp.float32)
    out_ref[:, _FEAT:] = ones_col


def _tc_edges(ea, xs, w1, b1t, w2aug):
    grid = _N_EDGES // _BE
    return pl.pallas_call(
        _edge_body,
        grid=(grid,),
        in_specs=[
            pl.BlockSpec((_BE, _EDGE_DIM), lambda i: (i, 0)),
            pl.BlockSpec((_BE, _DP), lambda i: (i, 0)),
            pl.BlockSpec((_EDGE_DIM, _EHID), lambda i: (0, 0)),
            pl.BlockSpec((_EHID, 1), lambda i: (0, 0)),
            pl.BlockSpec((_DP, _KN + _FEAT), lambda i: (0, 0)),
        ],
        out_specs=pl.BlockSpec((_BE, _DP), lambda i: (i, 0)),
        out_shape=jax.ShapeDtypeStruct((_N_EDGES, _DP), jnp.float32),
    )(ea, xs, w1, b1t, w2aug)


def _node_body(x1_ref, p_ref, rw_ref, b_ref, out_ref):
    s = p_ref[0] + p_ref[1]
    agg = s[:, :_FEAT] / jnp.maximum(s[:, _FEAT:_FEAT + 1], 1.0)
    out = (jnp.dot(x1_ref[...], rw_ref[...], preferred_element_type=jnp.float32)
           + agg + b_ref[...])
    out_ref[...] = jnp.maximum(out, 0.0)


def _tc_nodes(x1p, parts, rootp, bias2):
    grid = _N_NODES // _BN
    return pl.pallas_call(
        _node_body,
        grid=(grid,),
        in_specs=[
            pl.BlockSpec((_BN, _DP), lambda i: (i, 0)),
            pl.BlockSpec((_NC, _BN, _DP), lambda i: (0, i, 0)),
            pl.BlockSpec((_DP, _FEAT), lambda i: (0, 0)),
            pl.BlockSpec((1, _FEAT), lambda i: (0, 0)),
        ],
        out_specs=pl.BlockSpec((_BN, _FEAT), lambda i: (i, 0)),
        out_shape=jax.ShapeDtypeStruct((_N_NODES, _FEAT), jnp.float32),
    )(x1p, parts, rootp, bias2)


@jax.jit
def kernel(x, edge_index, edge_attr, lin1_w, lin1_b, lin2_w, lin2_b,
           root_w, bias):
    x1p = jnp.pad(x[:, :_HI_B], ((0, 0), (0, _DP - _HI_B)))
    src = edge_index[0]
    dst = edge_index[1]
    w2r = jnp.pad(
        lin2_w.reshape(_EHID, _HI_B, _FEAT).transpose(1, 0, 2)
        .reshape(_HI_B, _KN), ((0, _DP - _HI_B), (0, 0)))
    b2p = jnp.pad(lin2_b.reshape(_HI_B, _FEAT), ((0, _DP - _HI_B), (0, 0)))
    w2aug = jnp.concatenate([w2r, b2p], axis=1)
    rootp = jnp.pad(root_w, ((0, _DP - _HI_B), (0, 0)))

    x_src = _sc_gather(x1p, src)
    msg = _tc_edges(edge_attr, x_src, lin1_w, lin1_b.reshape(-1, 1), w2aug)
    parts = _sc_scatter(msg, dst, jnp.zeros((_N_NODES, _DP), jnp.float32))
    return _tc_nodes(x1p, parts, rootp, bias.reshape(1, -1))
